# baseline (speedup 1.0000x reference)
import functools
import math

import jax
import jax.numpy as jnp
from jax import lax
from jax.experimental import pallas as pl
from jax.experimental.pallas import tpu as pltpu
from jax.experimental.pallas import tpu_sc as plsc

M_EPSILON = 1e-05
_HALF_PI = math.pi / 2.0
_S0, _S1, _S2, _S3 = (0.9999966, -0.16664824, 0.008306286, -0.00018362749)
_C0, _C1, _C2, _C3, _C4 = (0.99999994, -0.49999905, 0.04166358, -0.0013853667, 2.3153174e-05)


def _sin_poly(x, x2):
    return x * (_S0 + x2 * (_S1 + x2 * (_S2 + x2 * _S3)))


def _cos_poly(x2):
    return _C0 + x2 * (_C1 + x2 * (_C2 + x2 * (_C3 + x2 * _C4)))


_B = 16384
_K = 64
_NW = 32
_CHUNK = 128
_COLS_PER_W = _B // _NW


def kernel(lam, kappa, theta, phi, wi):
    b, k = lam.shape
    lam_t = lam.T
    kappa_t = kappa.T
    theta_t = theta.T
    phi_t = phi.T
    w0 = wi[:, 0]
    w1 = wi[:, 1]
    w2 = wi[:, 2]

    mesh = plsc.VectorSubcoreMesh(core_axis_name="c", subcore_axis_name="s")

    @functools.partial(
        pl.kernel,
        mesh=mesh,
        out_type=jax.ShapeDtypeStruct((b,), jnp.float32),
        scratch_types=[
            pltpu.VMEM((_K, _CHUNK), jnp.float32),
            pltpu.VMEM((_K, _CHUNK), jnp.float32),
            pltpu.VMEM((_K, _CHUNK), jnp.float32),
            pltpu.VMEM((_K, _CHUNK), jnp.float32),
            pltpu.VMEM((_CHUNK,), jnp.float32),
            pltpu.VMEM((_CHUNK,), jnp.float32),
            pltpu.VMEM((_CHUNK,), jnp.float32),
            pltpu.VMEM((_CHUNK,), jnp.float32),
        ],
    )
    def sck(lam_h, kappa_h, theta_h, phi_h, w0_h, w1_h, w2_h, out_h,
            lam_v, kappa_v, theta_v, phi_v, w0_v, w1_v, w2_v, out_v):
        wid = lax.axis_index("s") * 2 + lax.axis_index("c")
        base = wid * _COLS_PER_W
        for cc in range(_COLS_PER_W // _CHUNK):
            col0 = base + cc * _CHUNK
            pltpu.sync_copy(lam_h.at[:, pl.ds(col0, _CHUNK)], lam_v)
            pltpu.sync_copy(kappa_h.at[:, pl.ds(col0, _CHUNK)], kappa_v)
            pltpu.sync_copy(theta_h.at[:, pl.ds(col0, _CHUNK)], theta_v)
            pltpu.sync_copy(phi_h.at[:, pl.ds(col0, _CHUNK)], phi_v)
            pltpu.sync_copy(w0_h.at[pl.ds(col0, _CHUNK)], w0_v)
            pltpu.sync_copy(w1_h.at[pl.ds(col0, _CHUNK)], w1_v)
            pltpu.sync_copy(w2_h.at[pl.ds(col0, _CHUNK)], w2_v)
            for g in range(_CHUNK // 16):
                w0r = w0_v[pl.ds(g * 16, 16)]
                w1r = w1_v[pl.ds(g * 16, 16)]
                w2r = w2_v[pl.ds(g * 16, 16)]

                def body(kk, carry):
                    num, den = carry
                    lam16 = lam_v[kk, pl.ds(g * 16, 16)]
                    kap16 = kappa_v[kk, pl.ds(g * 16, 16)]
                    th16 = theta_v[kk, pl.ds(g * 16, 16)]
                    ph16 = phi_v[kk, pl.ds(g * 16, 16)]

                    lambdas = jnp.maximum(lam16, 0.0) + 1e-06
                    x = th16 - _HALF_PI
                    x2 = x * x
                    st = _cos_poly(x2)
                    nct = _sin_poly(x, x2)
                    y = ph16 * 0.5 - _HALF_PI
                    y2 = y * y
                    sy = _sin_poly(y, y2)
                    cy = _cos_poly(y2)
                    nsp = 2.0 * sy * cy
                    cp = 2.0 * (sy * sy) - 1.0
                    dots = st * (cp * w0r - nsp * w1r) - nct * w2r
                    safe = jnp.maximum(kap16, 1e-06)
                    denom = (2.0 * math.pi) * (1.0 - jnp.exp(-2.0 * safe))
                    num = num + lambdas * ((safe / denom) * jnp.exp(kap16 * (dots - 1.0)))
                    den = den + lambdas
                    return num, den

                num, den = lax.fori_loop(
                    0, _K, body,
                    (jnp.zeros((16,), jnp.float32), jnp.zeros((16,), jnp.float32)),
                )
                out_v[pl.ds(g * 16, 16)] = num / jnp.maximum(den, M_EPSILON)
            pltpu.sync_copy(out_v, out_h.at[pl.ds(col0, _CHUNK)])

    return sck(lam_t, kappa_t, theta_t, phi_t, w0, w1, w2)


# speedup vs baseline: 2.4099x; 2.4099x over previous
import functools
import math

import jax
import jax.numpy as jnp
from jax import lax
from jax.experimental import pallas as pl
from jax.experimental.pallas import tpu as pltpu
from jax.experimental.pallas import tpu_sc as plsc

M_EPSILON = 1e-05
_HALF_PI = math.pi / 2.0

_S0, _S1, _S2, _S3 = (0.9999966, -0.16664824, 0.008306286, -0.00018362749)
_C0, _C1, _C2, _C3, _C4 = (
    0.99999994,
    -0.49999905,
    0.04166358,
    -0.0013853667,
    2.3153174e-05,
)


def _sin_poly(x, x2):
    return x * (_S0 + x2 * (_S1 + x2 * (_S2 + x2 * _S3)))


def _cos_poly(x2):
    return _C0 + x2 * (_C1 + x2 * (_C2 + x2 * (_C3 + x2 * _C4)))


def _vmf_terms(lam, kappa, theta, phi, w0, w1, w2):
    lambdas = jnp.maximum(lam, 0.0) + 1e-06

    x = theta - _HALF_PI
    x2 = x * x
    st = _cos_poly(x2)
    nct = _sin_poly(x, x2)

    y = phi * 0.5 - _HALF_PI
    y2 = y * y
    sy = _sin_poly(y, y2)
    cy = _cos_poly(y2)
    nsp = 2.0 * sy * cy
    cp = 2.0 * (sy * sy) - 1.0

    dots = st * (cp * w0 - nsp * w1) - nct * w2

    safe = jnp.maximum(kappa, 1e-06)
    denom = (2.0 * math.pi) * (1.0 - jnp.exp(-2.0 * safe))
    return lambdas * ((safe / denom) * jnp.exp(kappa * (dots - 1.0))), lambdas


_BLOCK_L = 2048
_KT = 8


def _tc_body(lam_ref, kappa_ref, theta_ref, phi_ref, w0_ref, w1_ref, w2_ref, out_ref):
    w0 = w0_ref[...][None, :]
    w1 = w1_ref[...][None, :]
    w2 = w2_ref[...][None, :]

    k = lam_ref.shape[0]
    num8 = jnp.zeros((_KT, _BLOCK_L), jnp.float32)
    den8 = jnp.zeros((_KT, _BLOCK_L), jnp.float32)
    for t in range(k // _KT):
        sl = slice(t * _KT, (t + 1) * _KT)
        numt, dent = _vmf_terms(
            lam_ref[sl, :], kappa_ref[sl, :], theta_ref[sl, :], phi_ref[sl, :],
            w0, w1, w2,
        )
        num8 = num8 + numt
        den8 = den8 + dent

    num = jnp.sum(num8, axis=0)
    den = jnp.maximum(jnp.sum(den8, axis=0), M_EPSILON)
    out_ref[...] = num / den


_NW = 32
_SC_COLS = 2048
_CHUNK = _SC_COLS // _NW


def _make_sc_kernel(b, k):
    mesh = plsc.VectorSubcoreMesh(core_axis_name="c", subcore_axis_name="s")

    @functools.partial(
        pl.kernel,
        mesh=mesh,
        out_type=jax.ShapeDtypeStruct((_SC_COLS,), jnp.float32),
        scratch_types=[
            pltpu.VMEM((k, 2 * _CHUNK), jnp.float32),
            pltpu.VMEM((k, 2 * _CHUNK), jnp.float32),
            pltpu.VMEM((k, 2 * _CHUNK), jnp.float32),
            pltpu.VMEM((k, 2 * _CHUNK), jnp.float32),
            pltpu.VMEM((_CHUNK,), jnp.float32),
            pltpu.VMEM((_CHUNK,), jnp.float32),
            pltpu.VMEM((_CHUNK,), jnp.float32),
            pltpu.VMEM((_CHUNK,), jnp.float32),
        ],
    )
    def sck(lam_h, kappa_h, theta_h, phi_h, w0_h, w1_h, w2_h, out_h,
            lam_v, kappa_v, theta_v, phi_v, w0_v, w1_v, w2_v, out_v):
        wid = lax.axis_index("s") * 2 + lax.axis_index("c")
        slab0 = (wid // 2) * (2 * _CHUNK)
        half = (wid % 2) * _CHUNK
        col0 = slab0 + half
        pltpu.sync_copy(lam_h.at[:, pl.ds(slab0, 2 * _CHUNK)], lam_v)
        pltpu.sync_copy(kappa_h.at[:, pl.ds(slab0, 2 * _CHUNK)], kappa_v)
        pltpu.sync_copy(theta_h.at[:, pl.ds(slab0, 2 * _CHUNK)], theta_v)
        pltpu.sync_copy(phi_h.at[:, pl.ds(slab0, 2 * _CHUNK)], phi_v)
        pltpu.sync_copy(w0_h.at[pl.ds(col0, _CHUNK)], w0_v)
        pltpu.sync_copy(w1_h.at[pl.ds(col0, _CHUNK)], w1_v)
        pltpu.sync_copy(w2_h.at[pl.ds(col0, _CHUNK)], w2_v)
        for gg in range(_CHUNK // 16):
            w0r = w0_v[pl.ds(gg * 16, 16)]
            w1r = w1_v[pl.ds(gg * 16, 16)]
            w2r = w2_v[pl.ds(gg * 16, 16)]

            def body(kk, carry):
                num, den = carry
                numt, dent = _vmf_terms(
                    lam_v[kk, pl.ds(half + gg * 16, 16)],
                    kappa_v[kk, pl.ds(half + gg * 16, 16)],
                    theta_v[kk, pl.ds(half + gg * 16, 16)],
                    phi_v[kk, pl.ds(half + gg * 16, 16)],
                    w0r, w1r, w2r,
                )
                return num + numt, den + dent

            num, den = lax.fori_loop(
                0, k, body,
                (jnp.zeros((16,), jnp.float32), jnp.zeros((16,), jnp.float32)),
            )
            out_v[pl.ds(gg * 16, 16)] = num / jnp.maximum(den, M_EPSILON)
        pltpu.sync_copy(out_v, out_h.at[pl.ds(col0, _CHUNK)])

    return sck


def kernel(lam, kappa, theta, phi, wi):
    b, k = lam.shape
    lam_t = lam.T
    kappa_t = kappa.T
    theta_t = theta.T
    phi_t = phi.T
    w0 = wi[:, 0]
    w1 = wi[:, 1]
    w2 = wi[:, 2]

    sc_out = _make_sc_kernel(b, k)(lam_t, kappa_t, theta_t, phi_t, w0, w1, w2)

    tc_cols = b - _SC_COLS
    off = _SC_COLS // _BLOCK_L
    grid = (tc_cols // _BLOCK_L,)
    kb_spec = pl.BlockSpec((k, _BLOCK_L), lambda i: (0, i + off))
    w_spec = pl.BlockSpec((_BLOCK_L,), lambda i: (i + off,))

    tc_out = pl.pallas_call(
        _tc_body,
        grid=grid,
        in_specs=[kb_spec, kb_spec, kb_spec, kb_spec, w_spec, w_spec, w_spec],
        out_specs=pl.BlockSpec((_BLOCK_L,), lambda i: (i,)),
        out_shape=jax.ShapeDtypeStruct((tc_cols,), jnp.float32),
    )(lam_t, kappa_t, theta_t, phi_t, w0, w1, w2)

    return jnp.concatenate([sc_out, tc_out])


# speedup vs baseline: 5.4831x; 2.2752x over previous
import math

import jax
import jax.numpy as jnp
from jax.experimental import pallas as pl
from jax.experimental.pallas import tpu as pltpu

M_EPSILON = 1e-05
_BLOCK_L = 4096

_HALF_PI = math.pi / 2.0

_S0, _S1, _S2, _S3 = (0.9999966, -0.16664824, 0.008306286, -0.00018362749)
_C0, _C1, _C2, _C3, _C4 = (
    0.99999994,
    -0.49999905,
    0.04166358,
    -0.0013853667,
    2.3153174e-05,
)


def _sin_poly(x, x2):
    return x * (_S0 + x2 * (_S1 + x2 * (_S2 + x2 * _S3)))


def _cos_poly(x2):
    return _C0 + x2 * (_C1 + x2 * (_C2 + x2 * (_C3 + x2 * _C4)))


_KT = 8


def _body(lam_ref, kappa_ref, theta_ref, phi_ref, w0_ref, w1_ref, w2_ref, out_ref):
    w0 = w0_ref[...][None, :]
    w1 = w1_ref[...][None, :]
    w2 = w2_ref[...][None, :]

    k = lam_ref.shape[0]
    num8 = jnp.zeros((_KT, _BLOCK_L), jnp.float32)
    den8 = jnp.zeros((_KT, _BLOCK_L), jnp.float32)
    for t in range(k // _KT):
        sl = slice(t * _KT, (t + 1) * _KT)
        lam = lam_ref[sl, :]
        kappa = kappa_ref[sl, :]
        theta = theta_ref[sl, :]
        phi = phi_ref[sl, :]

        lambdas = jnp.maximum(lam, 0.0) + 1e-06

        x = theta - _HALF_PI
        x2 = x * x
        st = _cos_poly(x2)
        nct = _sin_poly(x, x2)

        y = phi * 0.5 - _HALF_PI
        y2 = y * y
        sy = _sin_poly(y, y2)
        cy = _cos_poly(y2)
        nsp = 2.0 * sy * cy
        cp = 2.0 * (sy * sy) - 1.0

        dots = st * (cp * w0 - nsp * w1) - nct * w2

        safe = jnp.maximum(kappa, 1e-06)
        denom = (2.0 * math.pi) * (1.0 - jnp.exp(-2.0 * safe))
        num8 = num8 + lambdas * ((safe / denom) * jnp.exp(kappa * (dots - 1.0)))
        den8 = den8 + lambdas

    num = jnp.sum(num8, axis=0)
    den = jnp.maximum(jnp.sum(den8, axis=0), M_EPSILON)
    out_ref[...] = num / den


def kernel(lam, kappa, theta, phi, wi):
    b, k = lam.shape
    lam_t = lam.T
    kappa_t = kappa.T
    theta_t = theta.T
    phi_t = phi.T
    w0 = wi[:, 0]
    w1 = wi[:, 1]
    w2 = wi[:, 2]

    grid = (b // _BLOCK_L,)
    kb_spec = pl.BlockSpec((k, _BLOCK_L), lambda i: (0, i))
    w_spec = pl.BlockSpec((_BLOCK_L,), lambda i: (i,))

    def outer(lam_h, kappa_h, theta_h, phi_h, w0_h, w1_h, w2_h, out_h):
        pltpu.emit_pipeline(
            _body,
            grid=grid,
            in_specs=[kb_spec, kb_spec, kb_spec, kb_spec, w_spec, w_spec, w_spec],
            out_specs=[pl.BlockSpec((_BLOCK_L,), lambda i: (i,))],
        )(lam_h, kappa_h, theta_h, phi_h, w0_h, w1_h, w2_h, out_h)

    return pl.pallas_call(
        outer,
        in_specs=[pl.BlockSpec(memory_space=pl.ANY)] * 7,
        out_specs=pl.BlockSpec(memory_space=pl.ANY),
        out_shape=jax.ShapeDtypeStruct((b,), jnp.float32),
    )(lam_t, kappa_t, theta_t, phi_t, w0, w1, w2)
